# Initial kernel scaffold; baseline (speedup 1.0000x reference)
import functools

import jax
import jax.numpy as jnp
from jax import lax
from jax.experimental import pallas as pl
from jax.experimental.pallas import tpu as pltpu
from jax.experimental.pallas import tpu_sc as plsc

_N_NODES = 10000
_N_EDGES = 320000
_DIM = 128
_BATCH = 4096

_NC = 2
_NS = 16
_NW = _NC * _NS
_L = 16

_EB = 128
_NB = -(-_N_EDGES // (_NW * _EB))
_EPW = _NB * _EB
_EPAD = _NW * _EPW
_R = 10240
_RPS = _R // _NS
_DEGW = 16
_SB = _BATCH // _NW
_BR = 1024

_mesh = plsc.VectorSubcoreMesh(core_axis_name="c", subcore_axis_name="s")


@functools.partial(
    pl.kernel,
    mesh=_mesh,
    out_type=[
        jax.ShapeDtypeStruct((_NC, _R, _DIM), jnp.float32),
        jax.ShapeDtypeStruct((_NC, _R, _DEGW), jnp.float32),
    ],
    scratch_types=[
        pltpu.VMEM((_NB, _EB), jnp.int32),
        pltpu.VMEM((_NB, _EB), jnp.int32),
        pltpu.VMEM((_EB, _DIM), jnp.float32),
        pltpu.VMEM((_EB, _DEGW), jnp.float32),
        pltpu.VMEM_SHARED((_R, _DIM), jnp.float32),
        pltpu.VMEM_SHARED((_R, _DEGW), jnp.float32),
        pltpu.SemaphoreType.DMA,
    ],
)
def _aggregate(src_hbm, dst_hbm, table_hbm, zrow_hbm, zdeg_hbm,
               agg_out, deg_out,
               src_v, dst_v, rows_v, ones_v, agg_sh, deg_sh, sem):
    c = lax.axis_index("c")
    s = lax.axis_index("s")
    wid = c * _NS + s

    pltpu.sync_copy(zrow_hbm.at[pl.ds(s * _RPS, _RPS)],
                    agg_sh.at[pl.ds(s * _RPS, _RPS)])
    pltpu.sync_copy(zdeg_hbm.at[pl.ds(s * _RPS, _RPS)],
                    deg_sh.at[pl.ds(s * _RPS, _RPS)])
    pltpu.sync_copy(src_hbm.at[wid], src_v)
    pltpu.sync_copy(dst_hbm.at[wid], dst_v)
    for i in range(_EB):
        ones_v[i, :] = jnp.full((_DEGW,), 1.0, jnp.float32)
    plsc.subcore_barrier()

    def body(j, carry):
        pltpu.async_copy(table_hbm.at[src_v.at[j]], rows_v, sem).wait()
        pltpu.sync_copy(rows_v, agg_sh.at[dst_v.at[j]], add=True)
        pltpu.sync_copy(ones_v, deg_sh.at[dst_v.at[j]], add=True)
        return carry

    lax.fori_loop(0, _NB, body, 0)
    plsc.subcore_barrier()

    pltpu.sync_copy(agg_sh.at[pl.ds(s * _RPS, _RPS)],
                    agg_out.at[c, pl.ds(s * _RPS, _RPS)])
    pltpu.sync_copy(deg_sh.at[pl.ds(s * _RPS, _RPS)],
                    deg_out.at[c, pl.ds(s * _RPS, _RPS)])


def _x_body(agg_ref, degf_ref, w_ref, x_ref):
    a = agg_ref[0] + agg_ref[1]
    d = (degf_ref[0] + degf_ref[1]).reshape(_BR, _DEGW)
    deg = lax.slice(d, (0, 0), (_BR, 1))
    norm = 1.0 / jnp.maximum(deg, 1.0)
    x_ref[...] = jnp.tanh(
        jnp.dot(a * norm, w_ref[...], preferred_element_type=jnp.float32))


def _x_from_agg(agg_part, deg_flat, W):
    return pl.pallas_call(
        _x_body,
        grid=(_R // _BR,),
        in_specs=[
            pl.BlockSpec((_NC, _BR, _DIM), lambda i: (0, i, 0)),
            pl.BlockSpec((_NC, _BR * _DEGW), lambda i: (0, i)),
            pl.BlockSpec((_DIM, _DIM), lambda i: (0, 0)),
        ],
        out_specs=pl.BlockSpec((_BR, _DIM), lambda i: (i, 0)),
        out_shape=jax.ShapeDtypeStruct((_R, _DIM), jnp.float32),
    )(agg_part, deg_flat, W)


@functools.partial(
    pl.kernel,
    mesh=_mesh,
    out_type=jax.ShapeDtypeStruct((_BATCH, _DIM), jnp.float32),
    scratch_types=[
        pltpu.VMEM((_SB,), jnp.int32),
        pltpu.VMEM((_SB,), jnp.int32),
        pltpu.VMEM((_SB,), jnp.int32),
        pltpu.VMEM((_SB, _DIM), jnp.float32),
        pltpu.VMEM((_SB, _DIM), jnp.float32),
        pltpu.VMEM((_SB, _DIM), jnp.float32),
        pltpu.SemaphoreType.DMA,
    ],
)
def _score(x_hbm, rel_hbm, hi_hbm, ri_hbm, ti_hbm, out_hbm,
           hi_v, ri_v, ti_v, h_rows, r_rows, t_rows, sem):
    c = lax.axis_index("c")
    s = lax.axis_index("s")
    wid = c * _NS + s
    pltpu.sync_copy(hi_hbm.at[wid], hi_v)
    pltpu.sync_copy(ri_hbm.at[wid], ri_v)
    pltpu.sync_copy(ti_hbm.at[wid], ti_v)
    cp1 = pltpu.async_copy(x_hbm.at[hi_v], h_rows, sem)
    cp2 = pltpu.async_copy(rel_hbm.at[ri_v], r_rows, sem)
    cp3 = pltpu.async_copy(x_hbm.at[ti_v], t_rows, sem)
    cp1.wait()
    cp2.wait()
    cp3.wait()

    def body(i, carry):
        for k in range(_DIM // _L):
            sl = pl.ds(k * _L, _L)
            h_rows[i, sl] = h_rows[i, sl] * (r_rows[i, sl] * t_rows[i, sl])
        return carry

    lax.fori_loop(0, _SB, body, 0)
    pltpu.sync_copy(h_rows, out_hbm.at[pl.ds(wid * _SB, _SB)])


def kernel(sample, edge_index, edge_type, init_embed, init_rel, W):
    del edge_type
    src = edge_index[0]
    dst = edge_index[1]
    pad = _EPAD - _N_EDGES
    srcp = jnp.concatenate(
        [src, jnp.zeros((pad,), jnp.int32)]).reshape(_NW, _NB, _EB)
    dstp = jnp.concatenate(
        [dst, jnp.full((pad,), _N_NODES, jnp.int32)]).reshape(_NW, _NB, _EB)
    zrow = jnp.zeros((_R, _DIM), jnp.float32)
    zdeg = jnp.zeros((_R, _DEGW), jnp.float32)
    agg_part, deg_part = _aggregate(srcp, dstp, init_embed, zrow, zdeg)
    x = _x_from_agg(agg_part, deg_part.reshape(_NC, _R * _DEGW), W)
    hi = sample[:, 0].reshape(_NW, _SB)
    ri = sample[:, 1].reshape(_NW, _SB)
    ti = sample[:, 2].reshape(_NW, _SB)
    return _score(x, init_rel, hi, ri, ti)


# speedup vs baseline: 3.3646x; 3.3646x over previous
import functools

import jax
import jax.numpy as jnp
from jax import lax
from jax.experimental import pallas as pl
from jax.experimental.pallas import tpu as pltpu
from jax.experimental.pallas import tpu_sc as plsc

_N_NODES = 10000
_N_EDGES = 320000
_DIM = 128
_BATCH = 4096

_NC = 2
_NS = 16
_NW = _NC * _NS
_L = 16

_EB = 128
_CH = 4
_NB = 80
_EPW = _NB * _EB
_EPAD = _NW * _EPW
_R = 10240
_RPS = _R // _NS
_SB = _BATCH // _NW
_BR = 1024

_mesh = plsc.VectorSubcoreMesh(core_axis_name="c", subcore_axis_name="s")


@functools.partial(
    pl.kernel,
    mesh=_mesh,
    out_type=jax.ShapeDtypeStruct((_NC, _R, _DIM), jnp.float32),
    scratch_types=[
        pltpu.VMEM((_CH, _EB), jnp.int32),
        pltpu.VMEM((_CH, _EB), jnp.int32),
        pltpu.VMEM((_EB, _DIM), jnp.float32),
        pltpu.VMEM_SHARED((_R, _DIM), jnp.float32),
        pltpu.SemaphoreType.DMA,
    ],
)
def _aggregate(src_hbm, dst_hbm, table_hbm, zrow_hbm,
               agg_out,
               src_v, dst_v, rows_v, agg_sh, sem):
    c = lax.axis_index("c")
    s = lax.axis_index("s")
    wid = c * _NS + s

    pltpu.sync_copy(zrow_hbm, rows_v)
    for k in range(_RPS // _EB):
        pltpu.sync_copy(rows_v, agg_sh.at[pl.ds(s * _RPS + k * _EB, _EB)])
    plsc.subcore_barrier()

    def body(ci, carry):
        pltpu.sync_copy(src_hbm.at[wid, pl.ds(ci * _CH, _CH)], src_v)
        pltpu.sync_copy(dst_hbm.at[wid, pl.ds(ci * _CH, _CH)], dst_v)
        for b in range(_CH):
            pltpu.async_copy(table_hbm.at[src_v.at[b]], rows_v, sem).wait()
            pltpu.sync_copy(rows_v, agg_sh.at[dst_v.at[b]], add=True)
        return carry

    lax.fori_loop(0, _NB // _CH, body, 0)
    plsc.subcore_barrier()

    for k in range(_RPS // _EB):
        r0 = s * _RPS + k * _EB
        pltpu.sync_copy(agg_sh.at[pl.ds(r0, _EB)], rows_v)
        pltpu.sync_copy(rows_v, agg_out.at[c, pl.ds(r0, _EB)])


@functools.partial(
    pl.kernel,
    mesh=_mesh,
    out_type=jax.ShapeDtypeStruct((_NC, _R, _DIM), jnp.float32),
    scratch_types=[
        pltpu.VMEM((_CH, _EB), jnp.int32),
        pltpu.VMEM((_EB, _DIM), jnp.float32),
        pltpu.VMEM((_EB, _DIM), jnp.float32),
        pltpu.VMEM_SHARED((_R, _DIM), jnp.float32),
    ],
)
def _degree(dst_hbm, ones_hbm, zrow_hbm,
            deg_out,
            dst_v, ones_v, tmp_v, deg_sh):
    c = lax.axis_index("c")
    s = lax.axis_index("s")
    wid = c * _NS + s

    pltpu.sync_copy(ones_hbm, ones_v)
    pltpu.sync_copy(zrow_hbm, tmp_v)
    for k in range(_RPS // _EB):
        pltpu.sync_copy(tmp_v, deg_sh.at[pl.ds(s * _RPS + k * _EB, _EB)])
    plsc.subcore_barrier()

    def body(ci, carry):
        pltpu.sync_copy(dst_hbm.at[wid, pl.ds(ci * _CH, _CH)], dst_v)
        for b in range(_CH):
            pltpu.sync_copy(ones_v, deg_sh.at[dst_v.at[b]], add=True)
        return carry

    lax.fori_loop(0, _NB // _CH, body, 0)
    plsc.subcore_barrier()

    for k in range(_RPS // _EB):
        r0 = s * _RPS + k * _EB
        pltpu.sync_copy(deg_sh.at[pl.ds(r0, _EB)], tmp_v)
        pltpu.sync_copy(tmp_v, deg_out.at[c, pl.ds(r0, _EB)])


def _x_body(agg_ref, deg_ref, w_ref, x_ref):
    a = agg_ref[0] + agg_ref[1]
    d = deg_ref[0, :, 0:1] + deg_ref[1, :, 0:1]
    norm = 1.0 / jnp.maximum(d, 1.0)
    x_ref[...] = jnp.tanh(
        jnp.dot(a * norm, w_ref[...], preferred_element_type=jnp.float32))


def _x_from_agg(agg_part, deg_part, W):
    return pl.pallas_call(
        _x_body,
        grid=(_R // _BR,),
        in_specs=[
            pl.BlockSpec((_NC, _BR, _DIM), lambda i: (0, i, 0)),
            pl.BlockSpec((_NC, _BR, _DIM), lambda i: (0, i, 0)),
            pl.BlockSpec((_DIM, _DIM), lambda i: (0, 0)),
        ],
        out_specs=pl.BlockSpec((_BR, _DIM), lambda i: (i, 0)),
        out_shape=jax.ShapeDtypeStruct((_R, _DIM), jnp.float32),
    )(agg_part, deg_part, W)


@functools.partial(
    pl.kernel,
    mesh=_mesh,
    out_type=jax.ShapeDtypeStruct((_BATCH, _DIM), jnp.float32),
    scratch_types=[
        pltpu.VMEM((_SB,), jnp.int32),
        pltpu.VMEM((_SB,), jnp.int32),
        pltpu.VMEM((_SB,), jnp.int32),
        pltpu.VMEM((_SB, _DIM), jnp.float32),
        pltpu.VMEM((_SB, _DIM), jnp.float32),
        pltpu.VMEM((_SB, _DIM), jnp.float32),
        pltpu.SemaphoreType.DMA,
    ],
)
def _score(x_hbm, rel_hbm, hi_hbm, ri_hbm, ti_hbm, out_hbm,
           hi_v, ri_v, ti_v, h_rows, r_rows, t_rows, sem):
    c = lax.axis_index("c")
    s = lax.axis_index("s")
    wid = c * _NS + s
    pltpu.sync_copy(hi_hbm.at[wid], hi_v)
    pltpu.sync_copy(ri_hbm.at[wid], ri_v)
    pltpu.sync_copy(ti_hbm.at[wid], ti_v)
    cp1 = pltpu.async_copy(x_hbm.at[hi_v], h_rows, sem)
    cp2 = pltpu.async_copy(rel_hbm.at[ri_v], r_rows, sem)
    cp3 = pltpu.async_copy(x_hbm.at[ti_v], t_rows, sem)
    cp1.wait()
    cp2.wait()
    cp3.wait()

    def body(i, carry):
        for k in range(_DIM // _L):
            sl = pl.ds(k * _L, _L)
            h_rows[i, sl] = h_rows[i, sl] * (r_rows[i, sl] * t_rows[i, sl])
        return carry

    lax.fori_loop(0, _SB, body, 0)
    pltpu.sync_copy(h_rows, out_hbm.at[pl.ds(wid * _SB, _SB)])


def kernel(sample, edge_index, edge_type, init_embed, init_rel, W):
    del edge_type
    src = edge_index[0]
    dst = edge_index[1]
    pad = _EPAD - _N_EDGES
    srcp = jnp.concatenate(
        [src, jnp.zeros((pad,), jnp.int32)]).reshape(_NW, _NB, _EB)
    dstp = jnp.concatenate(
        [dst, jnp.full((pad,), _N_NODES, jnp.int32)]).reshape(_NW, _NB, _EB)
    zrow = jnp.zeros((_EB, _DIM), jnp.float32)
    agg_part = _aggregate(srcp, dstp, init_embed, zrow)
    dstp2, agg_part = lax.optimization_barrier((dstp, agg_part))
    ones_row = jnp.ones((_EB, _DIM), jnp.float32)
    deg_part = _degree(dstp2, ones_row, zrow)
    x = _x_from_agg(agg_part, deg_part, W)
    hi = sample[:, 0].reshape(_NW, _SB)
    ri = sample[:, 1].reshape(_NW, _SB)
    ti = sample[:, 2].reshape(_NW, _SB)
    return _score(x, init_rel, hi, ri, ti)


# speedup vs baseline: 3.7374x; 1.1108x over previous
import functools

import jax
import jax.numpy as jnp
from jax import lax
from jax.experimental import pallas as pl
from jax.experimental.pallas import tpu as pltpu
from jax.experimental.pallas import tpu_sc as plsc

_N_NODES = 10000
_N_EDGES = 320000
_DIM = 128
_BATCH = 4096

_NC = 2
_NS = 16
_NW = _NC * _NS
_L = 16

_EB = 128
_CH = 4
_NB = 80
_EPW = _NB * _EB
_EPAD = _NW * _EPW
_R = 10240
_RPS = _R // _NS
_SB = _BATCH // _NW
_BR = 1024

_mesh = plsc.VectorSubcoreMesh(core_axis_name="c", subcore_axis_name="s")


@functools.partial(
    pl.kernel,
    mesh=_mesh,
    out_type=jax.ShapeDtypeStruct((_NC, _R, _DIM), jnp.float32),
    scratch_types=[
        pltpu.VMEM((2, _CH, _EB), jnp.int32),
        pltpu.VMEM((2, _CH, _EB), jnp.int32),
        pltpu.VMEM((_EB, _DIM), jnp.float32),
        pltpu.VMEM((_EB, _DIM), jnp.float32),
        pltpu.VMEM_SHARED((_R, _DIM), jnp.float32),
        pltpu.SemaphoreType.DMA,
        pltpu.SemaphoreType.DMA,
    ],
)
def _aggregate(src_hbm, dst_hbm, table_hbm, zrow_hbm,
               agg_out,
               src_v, dst_v, rows0, rows1, agg_sh, sem0, sem1):
    c = lax.axis_index("c")
    s = lax.axis_index("s")
    wid = c * _NS + s
    ncH = _NB // _CH

    pltpu.sync_copy(zrow_hbm, rows0)
    for k in range(_RPS // _EB):
        pltpu.sync_copy(rows0, agg_sh.at[pl.ds(s * _RPS + k * _EB, _EB)])
    plsc.subcore_barrier()

    pltpu.sync_copy(src_hbm.at[wid, pl.ds(0, _CH)], src_v.at[0])
    pltpu.sync_copy(dst_hbm.at[wid, pl.ds(0, _CH)], dst_v.at[0])
    bufs = ((rows0, sem0), (rows1, sem1))
    pltpu.async_copy(table_hbm.at[src_v.at[0, 0]], rows0, sem0)
    pltpu.async_copy(table_hbm.at[src_v.at[0, 1]], rows1, sem1)

    @pl.loop(0, ncH)
    def _chunk(ci):
        p = lax.rem(ci, 2)
        q = 1 - p

        @pl.when(ci + 1 < ncH)
        def _():
            pltpu.sync_copy(src_hbm.at[wid, pl.ds((ci + 1) * _CH, _CH)],
                            src_v.at[q])
            pltpu.sync_copy(dst_hbm.at[wid, pl.ds((ci + 1) * _CH, _CH)],
                            dst_v.at[q])

        for b in range(_CH):
            rv, sm = bufs[b % 2]
            pltpu.make_async_copy(table_hbm.at[src_v.at[p, b]], rv, sm).wait()
            pltpu.sync_copy(rv, agg_sh.at[dst_v.at[p, b]], add=True)
            if b + 2 < _CH:
                pltpu.async_copy(table_hbm.at[src_v.at[p, b + 2]], rv, sm)
            else:

                @pl.when(ci + 1 < ncH)
                def _():
                    pltpu.async_copy(
                        table_hbm.at[src_v.at[q, b + 2 - _CH]], rv, sm)

    plsc.subcore_barrier()

    for k in range(_RPS // _EB):
        r0 = s * _RPS + k * _EB
        pltpu.sync_copy(agg_sh.at[pl.ds(r0, _EB)], rows0)
        pltpu.sync_copy(rows0, agg_out.at[c, pl.ds(r0, _EB)])


@functools.partial(
    pl.kernel,
    mesh=_mesh,
    out_type=jax.ShapeDtypeStruct((_NC, _R, _DIM), jnp.float32),
    scratch_types=[
        pltpu.VMEM((_CH, _EB), jnp.int32),
        pltpu.VMEM((_EB, _DIM), jnp.float32),
        pltpu.VMEM((_EB, _DIM), jnp.float32),
        pltpu.VMEM_SHARED((_R, _DIM), jnp.float32),
    ],
)
def _degree(dst_hbm, ones_hbm, zrow_hbm,
            deg_out,
            dst_v, ones_v, tmp_v, deg_sh):
    c = lax.axis_index("c")
    s = lax.axis_index("s")
    wid = c * _NS + s

    pltpu.sync_copy(ones_hbm, ones_v)
    pltpu.sync_copy(zrow_hbm, tmp_v)
    for k in range(_RPS // _EB):
        pltpu.sync_copy(tmp_v, deg_sh.at[pl.ds(s * _RPS + k * _EB, _EB)])
    plsc.subcore_barrier()

    def body(ci, carry):
        pltpu.sync_copy(dst_hbm.at[wid, pl.ds(ci * _CH, _CH)], dst_v)
        for b in range(_CH):
            pltpu.sync_copy(ones_v, deg_sh.at[dst_v.at[b]], add=True)
        return carry

    lax.fori_loop(0, _NB // _CH, body, 0)
    plsc.subcore_barrier()

    for k in range(_RPS // _EB):
        r0 = s * _RPS + k * _EB
        pltpu.sync_copy(deg_sh.at[pl.ds(r0, _EB)], tmp_v)
        pltpu.sync_copy(tmp_v, deg_out.at[c, pl.ds(r0, _EB)])


def _x_body(agg_ref, deg_ref, w_ref, x_ref):
    a = agg_ref[0] + agg_ref[1]
    d = deg_ref[0, :, 0:1] + deg_ref[1, :, 0:1]
    norm = 1.0 / jnp.maximum(d, 1.0)
    x_ref[...] = jnp.tanh(
        jnp.dot(a * norm, w_ref[...], preferred_element_type=jnp.float32))


def _x_from_agg(agg_part, deg_part, W):
    return pl.pallas_call(
        _x_body,
        grid=(_R // _BR,),
        in_specs=[
            pl.BlockSpec((_NC, _BR, _DIM), lambda i: (0, i, 0)),
            pl.BlockSpec((_NC, _BR, _DIM), lambda i: (0, i, 0)),
            pl.BlockSpec((_DIM, _DIM), lambda i: (0, 0)),
        ],
        out_specs=pl.BlockSpec((_BR, _DIM), lambda i: (i, 0)),
        out_shape=jax.ShapeDtypeStruct((_R, _DIM), jnp.float32),
    )(agg_part, deg_part, W)


@functools.partial(
    pl.kernel,
    mesh=_mesh,
    out_type=jax.ShapeDtypeStruct((_BATCH, _DIM), jnp.float32),
    scratch_types=[
        pltpu.VMEM((_SB,), jnp.int32),
        pltpu.VMEM((_SB,), jnp.int32),
        pltpu.VMEM((_SB,), jnp.int32),
        pltpu.VMEM((_SB, _DIM), jnp.float32),
        pltpu.VMEM((_SB, _DIM), jnp.float32),
        pltpu.VMEM((_SB, _DIM), jnp.float32),
        pltpu.SemaphoreType.DMA,
    ],
)
def _score(x_hbm, rel_hbm, hi_hbm, ri_hbm, ti_hbm, out_hbm,
           hi_v, ri_v, ti_v, h_rows, r_rows, t_rows, sem):
    c = lax.axis_index("c")
    s = lax.axis_index("s")
    wid = c * _NS + s
    pltpu.sync_copy(hi_hbm.at[wid], hi_v)
    pltpu.sync_copy(ri_hbm.at[wid], ri_v)
    pltpu.sync_copy(ti_hbm.at[wid], ti_v)
    cp1 = pltpu.async_copy(x_hbm.at[hi_v], h_rows, sem)
    cp2 = pltpu.async_copy(rel_hbm.at[ri_v], r_rows, sem)
    cp3 = pltpu.async_copy(x_hbm.at[ti_v], t_rows, sem)
    cp1.wait()
    cp2.wait()
    cp3.wait()

    def body(i, carry):
        for k in range(_DIM // _L):
            sl = pl.ds(k * _L, _L)
            h_rows[i, sl] = h_rows[i, sl] * (r_rows[i, sl] * t_rows[i, sl])
        return carry

    lax.fori_loop(0, _SB, body, 0)
    pltpu.sync_copy(h_rows, out_hbm.at[pl.ds(wid * _SB, _SB)])


def kernel(sample, edge_index, edge_type, init_embed, init_rel, W):
    del edge_type
    src = edge_index[0]
    dst = edge_index[1]
    pad = _EPAD - _N_EDGES
    srcp = jnp.concatenate(
        [src, jnp.zeros((pad,), jnp.int32)]).reshape(_NW, _NB, _EB)
    dstp = jnp.concatenate(
        [dst, jnp.full((pad,), _N_NODES, jnp.int32)]).reshape(_NW, _NB, _EB)
    zrow = jnp.zeros((_EB, _DIM), jnp.float32)
    agg_part = _aggregate(srcp, dstp, init_embed, zrow)
    dstp2, agg_part = lax.optimization_barrier((dstp, agg_part))
    ones_row = jnp.ones((_EB, _DIM), jnp.float32)
    deg_part = _degree(dstp2, ones_row, zrow)
    x = _x_from_agg(agg_part, deg_part, W)
    hi = sample[:, 0].reshape(_NW, _SB)
    ri = sample[:, 1].reshape(_NW, _SB)
    ti = sample[:, 2].reshape(_NW, _SB)
    return _score(x, init_rel, hi, ri, ti)


# speedup vs baseline: 8.9312x; 2.3897x over previous
import functools

import jax
import jax.numpy as jnp
from jax import lax
from jax.experimental import pallas as pl
from jax.experimental.pallas import tpu as pltpu
from jax.experimental.pallas import tpu_sc as plsc

_N_NODES = 10000
_N_EDGES = 320000
_DIM = 128
_BATCH = 4096

_NC = 2
_NS = 16
_NW = _NC * _NS
_L = 16

_EB = 128
_CH = 4
_NB = 80
_EPW = _NB * _EB
_EPAD = _NW * _EPW
_R = 10240
_RPS = _R // _NS
_SB = _BATCH // _NW
_BR = 1024

_mesh = plsc.VectorSubcoreMesh(core_axis_name="c", subcore_axis_name="s")


@functools.partial(
    pl.kernel,
    mesh=_mesh,
    out_type=jax.ShapeDtypeStruct((_NC, _R, _DIM), jnp.float32),
    scratch_types=[
        pltpu.VMEM((2, _CH, _EB), jnp.int32),
        pltpu.VMEM((2, _CH, _EB), jnp.int32),
        pltpu.VMEM((_EB, _DIM), jnp.float32),
        pltpu.VMEM((_EB, _DIM), jnp.float32),
        pltpu.VMEM_SHARED((_R, _DIM), jnp.float32),
        pltpu.SemaphoreType.DMA,
        pltpu.SemaphoreType.DMA,
    ],
)
def _aggregate(src_hbm, dst_hbm, table_hbm, zrow_hbm,
               agg_out,
               src_v, dst_v, rows0, rows1, agg_sh, sem0, sem1):
    c = lax.axis_index("c")
    s = lax.axis_index("s")
    wid = c * _NS + s
    ncH = _NB // _CH

    pltpu.sync_copy(zrow_hbm, rows0)
    for k in range(_RPS // _EB):
        pltpu.sync_copy(rows0, agg_sh.at[pl.ds(s * _RPS + k * _EB, _EB)])
    plsc.subcore_barrier()

    pltpu.sync_copy(src_hbm.at[wid, pl.ds(0, _CH)], src_v.at[0])
    pltpu.sync_copy(dst_hbm.at[wid, pl.ds(0, _CH)], dst_v.at[0])
    bufs = ((rows0, sem0), (rows1, sem1))
    pltpu.async_copy(table_hbm.at[src_v.at[0, 0]], rows0, sem0)
    pltpu.async_copy(table_hbm.at[src_v.at[0, 1]], rows1, sem1)

    @pl.loop(0, ncH)
    def _chunk(ci):
        p = lax.rem(ci, 2)
        q = 1 - p

        @pl.when(ci + 1 < ncH)
        def _():
            pltpu.sync_copy(src_hbm.at[wid, pl.ds((ci + 1) * _CH, _CH)],
                            src_v.at[q])
            pltpu.sync_copy(dst_hbm.at[wid, pl.ds((ci + 1) * _CH, _CH)],
                            dst_v.at[q])

        for b in range(_CH):
            rv, sm = bufs[b % 2]
            pltpu.make_async_copy(table_hbm.at[src_v.at[p, b]], rv, sm).wait()
            pltpu.sync_copy(rv, agg_sh.at[dst_v.at[p, b]], add=True)
            if b + 2 < _CH:
                pltpu.async_copy(table_hbm.at[src_v.at[p, b + 2]], rv, sm)
            else:

                @pl.when(ci + 1 < ncH)
                def _():
                    pltpu.async_copy(
                        table_hbm.at[src_v.at[q, b + 2 - _CH]], rv, sm)

    plsc.subcore_barrier()

    for k in range(_RPS // _EB):
        r0 = s * _RPS + k * _EB
        pltpu.sync_copy(agg_sh.at[pl.ds(r0, _EB)], rows0)
        pltpu.sync_copy(rows0, agg_out.at[c, pl.ds(r0, _EB)])


@functools.partial(
    pl.kernel,
    mesh=_mesh,
    out_type=jax.ShapeDtypeStruct((_NC, _R, _DIM), jnp.float32),
    scratch_types=[
        pltpu.VMEM((_CH, _EB), jnp.int32),
        pltpu.VMEM((_EB, _DIM), jnp.float32),
        pltpu.VMEM((_EB, _DIM), jnp.float32),
        pltpu.VMEM_SHARED((_R, _DIM), jnp.float32),
    ],
)
def _degree(dst_hbm, ones_hbm, zrow_hbm,
            deg_out,
            dst_v, ones_v, tmp_v, deg_sh):
    c = lax.axis_index("c")
    s = lax.axis_index("s")
    wid = c * _NS + s

    pltpu.sync_copy(ones_hbm, ones_v)
    pltpu.sync_copy(zrow_hbm, tmp_v)
    for k in range(_RPS // _EB):
        pltpu.sync_copy(tmp_v, deg_sh.at[pl.ds(s * _RPS + k * _EB, _EB)])
    plsc.subcore_barrier()

    def body(ci, carry):
        pltpu.sync_copy(dst_hbm.at[wid, pl.ds(ci * _CH, _CH)], dst_v)
        for b in range(_CH):
            pltpu.sync_copy(ones_v, deg_sh.at[dst_v.at[b]], add=True)
        return carry

    lax.fori_loop(0, _NB // _CH, body, 0)
    plsc.subcore_barrier()

    for k in range(_RPS // _EB):
        r0 = s * _RPS + k * _EB
        pltpu.sync_copy(deg_sh.at[pl.ds(r0, _EB)], tmp_v)
        pltpu.sync_copy(tmp_v, deg_out.at[c, pl.ds(r0, _EB)])


def _x_body(agg_ref, deg_ref, w_ref, x_ref):
    a = agg_ref[0] + agg_ref[1]
    d = deg_ref[0, :, 0:1] + deg_ref[1, :, 0:1]
    norm = 1.0 / jnp.maximum(d, 1.0)
    x_ref[...] = jnp.tanh(
        jnp.dot(a * norm, w_ref[...], preferred_element_type=jnp.float32))


def _x_from_agg(agg_part, deg_part, W):
    return pl.pallas_call(
        _x_body,
        grid=(_R // _BR,),
        in_specs=[
            pl.BlockSpec((_NC, _BR, _DIM), lambda i: (0, i, 0)),
            pl.BlockSpec((_NC, _BR, _DIM), lambda i: (0, i, 0)),
            pl.BlockSpec((_DIM, _DIM), lambda i: (0, 0)),
        ],
        out_specs=pl.BlockSpec((_BR, _DIM), lambda i: (i, 0)),
        out_shape=jax.ShapeDtypeStruct((_R, _DIM), jnp.float32),
    )(agg_part, deg_part, W)


@functools.partial(
    pl.kernel,
    mesh=_mesh,
    out_type=jax.ShapeDtypeStruct((_BATCH, _DIM), jnp.float32),
    scratch_types=[
        pltpu.VMEM((_SB,), jnp.int32),
        pltpu.VMEM((_SB,), jnp.int32),
        pltpu.VMEM((_SB,), jnp.int32),
        pltpu.VMEM((_SB, _DIM), jnp.float32),
        pltpu.VMEM((_SB, _DIM), jnp.float32),
        pltpu.VMEM((_SB, _DIM), jnp.float32),
        pltpu.SemaphoreType.DMA,
    ],
)
def _score(x_hbm, rel_hbm, hi_hbm, ri_hbm, ti_hbm, out_hbm,
           hi_v, ri_v, ti_v, h_rows, r_rows, t_rows, sem):
    c = lax.axis_index("c")
    s = lax.axis_index("s")
    wid = c * _NS + s
    pltpu.sync_copy(hi_hbm.at[wid], hi_v)
    pltpu.sync_copy(ri_hbm.at[wid], ri_v)
    pltpu.sync_copy(ti_hbm.at[wid], ti_v)
    cp1 = pltpu.async_copy(x_hbm.at[hi_v], h_rows, sem)
    cp2 = pltpu.async_copy(rel_hbm.at[ri_v], r_rows, sem)
    cp3 = pltpu.async_copy(x_hbm.at[ti_v], t_rows, sem)
    cp1.wait()
    cp2.wait()
    cp3.wait()

    def body(i, carry):
        for k in range(_DIM // _L):
            sl = pl.ds(k * _L, _L)
            h_rows[i, sl] = h_rows[i, sl] * (r_rows[i, sl] * t_rows[i, sl])
        return carry

    lax.fori_loop(0, _SB, body, 0)
    pltpu.sync_copy(h_rows, out_hbm.at[pl.ds(wid * _SB, _SB)])


def kernel(sample, edge_index, edge_type, init_embed, init_rel, W):
    del edge_type
    src = edge_index[0]
    dst = edge_index[1]
    ppw = _EPW - _N_EDGES // _NW
    pad_src = jnp.broadcast_to(
        (jnp.arange(ppw, dtype=jnp.int32) * 37) % _N_NODES, (_NW, ppw))
    pad_dst = jnp.broadcast_to(
        _N_NODES + jnp.arange(ppw, dtype=jnp.int32) % (_R - _N_NODES),
        (_NW, ppw))
    srcp = jnp.concatenate(
        [src.reshape(_NW, _N_EDGES // _NW), pad_src],
        axis=1).reshape(_NW, _NB, _EB)
    dstp = jnp.concatenate(
        [dst.reshape(_NW, _N_EDGES // _NW), pad_dst],
        axis=1).reshape(_NW, _NB, _EB)
    zrow = jnp.zeros((_EB, _DIM), jnp.float32)
    agg_part = _aggregate(srcp, dstp, init_embed, zrow)
    dstp2, agg_part = lax.optimization_barrier((dstp, agg_part))
    ones_row = jnp.ones((_EB, _DIM), jnp.float32)
    deg_part = _degree(dstp2, ones_row, zrow)
    x = _x_from_agg(agg_part, deg_part, W)
    hi = sample[:, 0].reshape(_NW, _SB)
    ri = sample[:, 1].reshape(_NW, _SB)
    ti = sample[:, 2].reshape(_NW, _SB)
    return _score(x, init_rel, hi, ri, ti)
